# Initial kernel scaffold; baseline (speedup 1.0000x reference)
import jax
import jax.numpy as jnp
from jax.experimental import pallas as pl


def kernel(z_e, weight):
    raise NotImplementedError("write your pallas kernel here")


# speedup vs baseline: 2.6829x; 2.6829x over previous
import jax
import jax.numpy as jnp
from jax.experimental import pallas as pl

_K = 512
_D = 64


def _vq_body(z_ref, w_ref, wt_ref, out_ref):
    x = z_ref[0]
    w = w_ref[...]
    wt = wt_ref[...]
    hw = x.shape[1]

    wn = jnp.sum(w * w, axis=1, keepdims=True)
    zn = jnp.sum(x * x, axis=0, keepdims=True)
    scores = jax.lax.dot_general(
        w, x, dimension_numbers=(((1,), (0,)), ((), ())),
        preferred_element_type=jnp.float32)
    d = (zn + wn) - 2.0 * scores

    m = jnp.min(d, axis=0, keepdims=True)
    rows = jax.lax.broadcasted_iota(jnp.int32, (_K, hw), 0)
    idx = jnp.min(jnp.where(d <= m, rows, _K), axis=0)

    onehot = (rows == idx[None, :]).astype(jnp.float32)
    out_ref[0] = jax.lax.dot_general(
        wt, onehot, dimension_numbers=(((1,), (0,)), ((), ())),
        preferred_element_type=jnp.float32)


def kernel(z_e, weight):
    B, C, H, W = z_e.shape
    hw = H * W
    z = z_e.reshape(B, C, hw)
    wt = jnp.transpose(weight, (1, 0))

    out = pl.pallas_call(
        _vq_body,
        grid=(B,),
        in_specs=[
            pl.BlockSpec((1, C, hw), lambda b: (b, 0, 0)),
            pl.BlockSpec((_K, _D), lambda b: (0, 0)),
            pl.BlockSpec((_D, _K), lambda b: (0, 0)),
        ],
        out_specs=pl.BlockSpec((1, C, hw), lambda b: (b, 0, 0)),
        out_shape=jax.ShapeDtypeStruct((B, C, hw), jnp.float32),
    )(z, weight, wt)
    return out.reshape(B, C, H, W)
